# baseline (speedup 1.0000x reference)
import functools

import jax
import jax.numpy as jnp
from jax import lax
from jax.experimental import pallas as pl
from jax.experimental.pallas import tpu as pltpu
from jax.experimental.pallas import tpu_sc as plsc

D = 1024
H = 2048
E = 8
TOPK = 2
T = 2048
P = T * TOPK
BM = 256
NPAD = P + E * BM
NB = NPAD // BM
MMAX = 64
NACTO, RSO, SLO, NXTO = 63, 64, 128, 192
CHUNK = 512

NC = 2
NS = 16
NW = NC * NS
PPW = P // NW
CH = 32
NCHUNK = PPW // CH


def _router_body(x_ref, gw_ref, wts_ref, dest_ref, meta_ref, xp_ref):
    xx = x_ref[...]
    gw = gw_ref[...]
    lt = lax.dot_general(gw, xx, (((1,), (1,)), ((), ())),
                         preferred_element_type=jnp.float32)
    rowe = lax.broadcasted_iota(jnp.int32, (E, T), 0)
    m1 = jnp.max(lt, axis=0, keepdims=True)
    i1 = jnp.min(jnp.where(lt == m1, rowe, E), axis=0, keepdims=True)
    masked = jnp.where(rowe == i1, -jnp.inf, lt)
    m2 = jnp.max(masked, axis=0, keepdims=True)
    i2 = jnp.min(jnp.where(masked == m2, rowe, E), axis=0, keepdims=True)
    w1 = 1.0 / (1.0 + jnp.exp(m2 - m1))
    wts_ref[...] = jnp.concatenate([w1, 1.0 - w1], axis=0)

    ep = jnp.concatenate([i1, i2], axis=1)
    rowp = lax.broadcasted_iota(jnp.int32, (E, P), 0)
    onehot = (rowp == ep).astype(jnp.float32)

    r = lax.broadcasted_iota(jnp.int32, (CHUNK, CHUNK), 0)
    c = lax.broadcasted_iota(jnp.int32, (CHUNK, CHUNK), 1)
    triu = (r < c).astype(jnp.float32)
    totals = jnp.zeros((E, 1), jnp.float32)
    granks = []
    for ci in range(P // CHUNK):
        oc = onehot[:, ci * CHUNK:(ci + 1) * CHUNK]
        ranks = lax.dot_general(oc, triu, (((1,), (0,)), ((), ())),
                                preferred_element_type=jnp.float32) + totals
        granks.append(jnp.sum(oc * ranks, axis=0, keepdims=True))
        totals = totals + jnp.sum(oc, axis=1, keepdims=True)
    grank = jnp.concatenate(granks, axis=1)

    counts = totals.astype(jnp.int32)
    cpad = ((counts + BM - 1) // BM) * BM
    offs_rows = [jnp.zeros((1, 1), jnp.int32)]
    for e in range(1, E):
        offs_rows.append(offs_rows[-1] + cpad[e - 1:e])
    offs = jnp.concatenate(offs_rows, axis=0)

    dest = grank + jnp.sum(onehot * offs.astype(jnp.float32),
                           axis=0, keepdims=True)
    dest_ref[...] = dest.astype(jnp.int32)

    nact = jnp.sum(cpad) // BM
    bcol = lax.broadcasted_iota(jnp.int32, (E, MMAX), 1)
    brows = jnp.minimum(bcol, nact - 1) * BM
    be = jnp.sum((brows >= offs).astype(jnp.int32), axis=0, keepdims=True) - 1
    colid = lax.broadcasted_iota(jnp.int32, (1, MMAX), 1)

    be_prev = jnp.concatenate(
        [jnp.full((1, 1), -1, jnp.int32), be[:, :MMAX - 1]], axis=1)
    rs = ((be != be_prev) & (colid < nact)).astype(jnp.int32)
    rr = lax.broadcasted_iota(jnp.int32, (MMAX, MMAX), 0)
    cc = lax.broadcasted_iota(jnp.int32, (MMAX, MMAX), 1)
    triu_incl = (rr <= cc).astype(jnp.float32)
    cs = lax.dot_general(rs.astype(jnp.float32), triu_incl,
                         (((1,), (0,)), ((), ())),
                         preferred_element_type=jnp.float32).astype(jnp.int32)
    slot = (cs - 1) & 1

    eyef = (lax.broadcasted_iota(jnp.int32, (E, E), 0) ==
            lax.broadcasted_iota(jnp.int32, (E, E), 1)).astype(jnp.float32)
    nef = (cpad > 0).astype(jnp.float32)
    neT = lax.dot_general(nef, eyef, (((0,), (0,)), ((), ())))
    rowI = lax.broadcasted_iota(jnp.int32, (E, E), 0)
    colI = lax.broadcasted_iota(jnp.int32, (E, E), 1)
    cand = jnp.where((colI > rowI) & (neT > 0), colI, 99)
    nxt_e = jnp.min(cand, axis=1, keepdims=True)
    ohbe = (lax.broadcasted_iota(jnp.int32, (E, MMAX), 0) == be)
    nxtb = jnp.sum(jnp.where(ohbe, nxt_e, 0), axis=0, keepdims=True)

    be_n = jnp.where(colid == NACTO, nact, be)
    meta_ref[...] = jnp.concatenate([be_n, rs, slot, nxtb], axis=1)

    xlob = lax.bitcast_convert_type(xx[:, :D // 2], jnp.int32)
    xhib = lax.bitcast_convert_type(xx[:, D // 2:], jnp.int32)
    rb = jnp.int32(0x8000)
    xp_ref[...] = (lax.shift_right_logical(xlob + rb, 16) |
                   ((xhib + rb) & jnp.int32(-65536)))


def _router_call(flat, gate_w):
    return pl.pallas_call(
        _router_body,
        out_shape=(
            jax.ShapeDtypeStruct((TOPK, T), jnp.float32),
            jax.ShapeDtypeStruct((1, P), jnp.int32),
            jax.ShapeDtypeStruct((1, 4 * MMAX), jnp.int32),
            jax.ShapeDtypeStruct((T, D // 2), jnp.int32),
        ),
    )(flat, gate_w)


def _sc_mesh():
    return plsc.VectorSubcoreMesh(core_axis_name="c", subcore_axis_name="s")


def _dispatch_body(x_hbm, dest_hbm, xs_hbm, d0, d1, d2, d3, bufa, bufb, sem):
    wid = lax.axis_index("s") * NC + lax.axis_index("c")
    base = wid * PPW
    tbase = lax.rem(base, T)
    dbufs = [d0, d1, d2, d3]
    rbufs = [bufa, bufb]
    handles = []
    for i in range(NCHUNK):
        buf = rbufs[i % 2]
        if i >= 2:
            handles[i - 2].wait()
        pltpu.sync_copy(x_hbm.at[pl.ds(tbase + i * CH, CH)], buf)
        pltpu.sync_copy(dest_hbm.at[pl.ds(base + i * CH, CH)], dbufs[i])
        handles.append(pltpu.async_copy(buf, xs_hbm.at[dbufs[i]], sem))
    handles[-2].wait()
    handles[-1].wait()


def _dispatch_call(xp, dest):
    f = functools.partial(
        pl.kernel,
        mesh=_sc_mesh(),
        out_type=jax.ShapeDtypeStruct((NPAD, D // 2), jnp.int32),
        scratch_types=[
            pltpu.VMEM((CH,), jnp.int32), pltpu.VMEM((CH,), jnp.int32),
            pltpu.VMEM((CH,), jnp.int32), pltpu.VMEM((CH,), jnp.int32),
            pltpu.VMEM((CH, D // 2), jnp.int32), pltpu.VMEM((CH, D // 2), jnp.int32),
            pltpu.SemaphoreType.DMA,
        ],
    )(_dispatch_body)
    return f(xp, dest)


TW = T // NW
CT = 16
NCT = TW // CT


def _cc_body(eo_hbm, dest_hbm, w0_hbm, w1_hbm, out_hbm,
             d0a, d1a, w0a, w1a, d0b, d1b, w0b, w1b,
             r0a, r1a, r0b, r1b, oba, obb, gsem, wsem):
    wid = lax.axis_index("s") * NC + lax.axis_index("c")
    tbase = wid * TW
    sets = [(d0a, d1a, w0a, w1a, r0a, r1a), (d0b, d1b, w0b, w1b, r0b, r1b)]
    obufs = [oba, obb]

    def load_idx(c):
        d0, d1, w0, w1, _, _ = sets[c % 2]
        tb = tbase + c * CT
        pltpu.sync_copy(dest_hbm.at[pl.ds(tb, CT)], d0)
        pltpu.sync_copy(dest_hbm.at[pl.ds(T + tb, CT)], d1)
        pltpu.sync_copy(w0_hbm.at[pl.ds(tb, CT)], w0)
        pltpu.sync_copy(w1_hbm.at[pl.ds(tb, CT)], w1)

    def issue_gathers(c):
        d0, d1, _, _, r0, r1 = sets[c % 2]
        return (pltpu.async_copy(eo_hbm.at[d0], r0, gsem),
                pltpu.async_copy(eo_hbm.at[d1], r1, gsem))

    load_idx(0)
    g = {0: issue_gathers(0)}
    whandles = {}
    for c in range(NCT):
        if c + 1 < NCT:
            load_idx(c + 1)
            g[c + 1] = issue_gathers(c + 1)
        g[c][0].wait()
        g[c][1].wait()
        if c >= 2:
            whandles[c - 2].wait()
        _, _, w0, w1, r0, r1 = sets[c % 2]
        ob = obufs[c % 2]

        def token_body(i, carry):
            idxv = jnp.full((16, 1), i, jnp.int32)
            gd = lax.GatherDimensionNumbers(offset_dims=(),
                                            collapsed_slice_dims=(0,),
                                            start_index_map=(0,))
            w0s = lax.gather(w0[...], idxv, gd, (1,),
                             mode=lax.GatherScatterMode.PROMISE_IN_BOUNDS)
            w1s = lax.gather(w1[...], idxv, gd, (1,),
                             mode=lax.GatherScatterMode.PROMISE_IN_BOUNDS)

            def lane_body(j, carry2):
                p0 = r0[i, pl.ds(j * 16, 16)]
                p1 = r1[i, pl.ds(j * 16, 16)]
                lo0 = lax.bitcast_convert_type(lax.shift_left(p0, 16),
                                               jnp.float32)
                hi0 = lax.bitcast_convert_type(p0 & jnp.int32(-65536),
                                               jnp.float32)
                lo1 = lax.bitcast_convert_type(lax.shift_left(p1, 16),
                                               jnp.float32)
                hi1 = lax.bitcast_convert_type(p1 & jnp.int32(-65536),
                                               jnp.float32)
                ob[i, pl.ds(j * 16, 16)] = lo0 * w0s + lo1 * w1s
                ob[i, pl.ds(D // 2 + j * 16, 16)] = hi0 * w0s + hi1 * w1s
                return carry2

            return lax.fori_loop(0, D // 32, lane_body, carry)

        lax.fori_loop(0, CT, token_body, 0)
        whandles[c] = pltpu.async_copy(
            ob, out_hbm.at[pl.ds(tbase + c * CT, CT)], wsem)
    whandles[NCT - 2].wait()
    whandles[NCT - 1].wait()


def _cc_call(eo, dest, w0arr, w1arr):
    f = functools.partial(
        pl.kernel,
        mesh=_sc_mesh(),
        out_type=jax.ShapeDtypeStruct((T, D), jnp.float32),
        scratch_types=[
            pltpu.VMEM((CT,), jnp.int32), pltpu.VMEM((CT,), jnp.int32),
            pltpu.VMEM((CT,), jnp.float32), pltpu.VMEM((CT,), jnp.float32),
            pltpu.VMEM((CT,), jnp.int32), pltpu.VMEM((CT,), jnp.int32),
            pltpu.VMEM((CT,), jnp.float32), pltpu.VMEM((CT,), jnp.float32),
            pltpu.VMEM((CT, D // 2), jnp.int32), pltpu.VMEM((CT, D // 2), jnp.int32),
            pltpu.VMEM((CT, D // 2), jnp.int32), pltpu.VMEM((CT, D // 2), jnp.int32),
            pltpu.VMEM((CT, D), jnp.float32), pltpu.VMEM((CT, D), jnp.float32),
            pltpu.SemaphoreType.DMA, pltpu.SemaphoreType.DMA,
        ],
    )(_cc_body)
    return f(eo, dest, w0arr, w1arr)


def _ffn_body(meta_ref, xs_ref, w1_any, w3_any, w2_any, out_ref,
              w1b0, w3b0, w2b0, w1b1, w3b1, w2b1, sem0, sem1):
    b = pl.program_id(0)
    nact = meta_ref[NACTO]

    def issue(e, bufs, sem):
        pltpu.make_async_copy(w1_any.at[e], bufs[0], sem).start()
        pltpu.make_async_copy(w3_any.at[e], bufs[1], sem).start()
        pltpu.make_async_copy(w2_any.at[e], bufs[2], sem).start()

    def drain(bufs, sem):
        pltpu.make_async_copy(w1_any.at[0], bufs[0], sem).wait()
        pltpu.make_async_copy(w3_any.at[0], bufs[1], sem).wait()
        pltpu.make_async_copy(w2_any.at[0], bufs[2], sem).wait()

    slot0 = (w1b0, w3b0, w2b0)
    slot1 = (w1b1, w3b1, w2b1)

    @pl.when(b == 0)
    def _():
        issue(meta_ref[0], slot0, sem0)
        nxt = meta_ref[NXTO]

        @pl.when(nxt < E)
        def _():
            issue(nxt, slot1, sem1)

        drain(slot0, sem0)

    @pl.when((b > 0) & (b < nact) & (meta_ref[RSO + b] == 1))
    def _():
        s = meta_ref[SLO + b]
        nxt = meta_ref[NXTO + b]

        @pl.when((nxt < E) & (s == 0))
        def _():
            issue(nxt, slot1, sem1)

        @pl.when((nxt < E) & (s == 1))
        def _():
            issue(nxt, slot0, sem0)

        @pl.when(s == 0)
        def _():
            drain(slot0, sem0)

        @pl.when(s == 1)
        def _():
            drain(slot1, sem1)

    def compute(bufs):
        xpk = xs_ref[...]
        xlo = lax.bitcast_convert_type(lax.shift_left(xpk, 16), jnp.float32)
        xhi = lax.bitcast_convert_type(xpk & jnp.int32(-65536), jnp.float32)
        xb = jnp.concatenate([xlo, xhi], axis=1)
        w1t = bufs[0][...]
        w3t = bufs[1][...]
        w2t = bufs[2][...]
        h1 = lax.dot_general(xb, w1t, (((1,), (1,)), ((), ())),
                             preferred_element_type=jnp.float32)
        h3 = lax.dot_general(xb, w3t, (((1,), (1,)), ((), ())),
                             preferred_element_type=jnp.float32)
        hh = h1 * (1.0 / (1.0 + jnp.exp(-h1))) * h3
        eo = lax.dot_general(hh, w2t, (((1,), (1,)), ((), ())),
                             preferred_element_type=jnp.float32)
        lob = lax.bitcast_convert_type(eo[:, :D // 2], jnp.int32)
        hib = lax.bitcast_convert_type(eo[:, D // 2:], jnp.int32)
        rb = jnp.int32(0x8000)
        lop = lax.shift_right_logical(lob + rb, 16)
        hip = (hib + rb) & jnp.int32(-65536)
        out_ref[...] = lop | hip

    @pl.when((b < nact) & (meta_ref[SLO + b] == 0))
    def _():
        compute(slot0)

    @pl.when((b < nact) & (meta_ref[SLO + b] == 1))
    def _():
        compute(slot1)


def _ffn_call(meta, xs, w1, w3, w2):
    grid_spec = pltpu.PrefetchScalarGridSpec(
        num_scalar_prefetch=1,
        grid=(NB,),
        in_specs=[
            pl.BlockSpec((BM, D // 2), lambda b, m: (b, 0)),
            pl.BlockSpec(memory_space=pl.ANY),
            pl.BlockSpec(memory_space=pl.ANY),
            pl.BlockSpec(memory_space=pl.ANY),
        ],
        out_specs=pl.BlockSpec((BM, D // 2), lambda b, m: (b, 0)),
        scratch_shapes=[
            pltpu.VMEM((H, D), jnp.float32), pltpu.VMEM((H, D), jnp.float32),
            pltpu.VMEM((D, H), jnp.float32),
            pltpu.VMEM((H, D), jnp.float32), pltpu.VMEM((H, D), jnp.float32),
            pltpu.VMEM((D, H), jnp.float32),
            pltpu.SemaphoreType.DMA, pltpu.SemaphoreType.DMA,
        ],
    )
    return pl.pallas_call(
        _ffn_body,
        grid_spec=grid_spec,
        out_shape=jax.ShapeDtypeStruct((NPAD, D // 2), jnp.int32),
        compiler_params=pltpu.CompilerParams(
            dimension_semantics=("arbitrary",),
        ),
    )(meta, xs, w1, w3, w2)


def kernel(x, gate_w, w1, w2, w3):
    B, T_, D_ = x.shape
    flat = x.reshape(T, D)
    wts, dest2, meta2, xp = _router_call(flat, gate_w)
    dest = dest2.reshape(P)
    meta = meta2.reshape(4 * MMAX)
    xs = _dispatch_call(xp, dest)
    eo = _ffn_call(meta, xs, w1, w3, w2)
    out = _cc_call(eo, dest, wts[0], wts[1])
    return out.reshape(B, T_, D_)


# speedup vs baseline: 1.0429x; 1.0429x over previous
import functools

import jax
import jax.numpy as jnp
from jax import lax
from jax.experimental import pallas as pl
from jax.experimental.pallas import tpu as pltpu
from jax.experimental.pallas import tpu_sc as plsc

D = 1024
H = 2048
E = 8
TOPK = 2
T = 2048
P = T * TOPK
BM = 256
NPAD = P + E * BM
NB = NPAD // BM
MMAX = 64
NACTO, RSO, SLO, NXTO = 63, 64, 128, 192
CHUNK = 512

NC = 2
NS = 16
NW = NC * NS
PPW = P // NW
CH = 32
NCHUNK = PPW // CH


def _router_body(x_ref, gw_ref, wts_ref, dest_ref, meta_ref, xp_ref):
    xx = x_ref[...]
    gw = gw_ref[...]
    lt = lax.dot_general(gw, xx, (((1,), (1,)), ((), ())),
                         preferred_element_type=jnp.float32)
    rowe = lax.broadcasted_iota(jnp.int32, (E, T), 0)
    m1 = jnp.max(lt, axis=0, keepdims=True)
    i1 = jnp.min(jnp.where(lt == m1, rowe, E), axis=0, keepdims=True)
    masked = jnp.where(rowe == i1, -jnp.inf, lt)
    m2 = jnp.max(masked, axis=0, keepdims=True)
    i2 = jnp.min(jnp.where(masked == m2, rowe, E), axis=0, keepdims=True)
    w1 = 1.0 / (1.0 + jnp.exp(m2 - m1))
    wts_ref[...] = jnp.concatenate([w1, 1.0 - w1], axis=0)

    ep = jnp.concatenate([i1, i2], axis=1)
    rowp = lax.broadcasted_iota(jnp.int32, (E, P), 0)
    onehot = (rowp == ep).astype(jnp.float32)

    r = lax.broadcasted_iota(jnp.int32, (CHUNK, CHUNK), 0)
    c = lax.broadcasted_iota(jnp.int32, (CHUNK, CHUNK), 1)
    triu = (r < c).astype(jnp.float32)
    totals = jnp.zeros((E, 1), jnp.float32)
    granks = []
    for ci in range(P // CHUNK):
        oc = onehot[:, ci * CHUNK:(ci + 1) * CHUNK]
        ranks = lax.dot_general(oc, triu, (((1,), (0,)), ((), ())),
                                preferred_element_type=jnp.float32) + totals
        granks.append(jnp.sum(oc * ranks, axis=0, keepdims=True))
        totals = totals + jnp.sum(oc, axis=1, keepdims=True)
    grank = jnp.concatenate(granks, axis=1)

    counts = totals.astype(jnp.int32)
    cpad = ((counts + BM - 1) // BM) * BM
    offs_rows = [jnp.zeros((1, 1), jnp.int32)]
    for e in range(1, E):
        offs_rows.append(offs_rows[-1] + cpad[e - 1:e])
    offs = jnp.concatenate(offs_rows, axis=0)

    dest = grank + jnp.sum(onehot * offs.astype(jnp.float32),
                           axis=0, keepdims=True)
    dest_ref[...] = dest.astype(jnp.int32)

    nact = jnp.sum(cpad) // BM
    bcol = lax.broadcasted_iota(jnp.int32, (E, MMAX), 1)
    brows = jnp.minimum(bcol, nact - 1) * BM
    be = jnp.sum((brows >= offs).astype(jnp.int32), axis=0, keepdims=True) - 1
    colid = lax.broadcasted_iota(jnp.int32, (1, MMAX), 1)

    be_prev = jnp.concatenate(
        [jnp.full((1, 1), -1, jnp.int32), be[:, :MMAX - 1]], axis=1)
    rs = ((be != be_prev) & (colid < nact)).astype(jnp.int32)
    rr = lax.broadcasted_iota(jnp.int32, (MMAX, MMAX), 0)
    cc = lax.broadcasted_iota(jnp.int32, (MMAX, MMAX), 1)
    triu_incl = (rr <= cc).astype(jnp.float32)
    cs = lax.dot_general(rs.astype(jnp.float32), triu_incl,
                         (((1,), (0,)), ((), ())),
                         preferred_element_type=jnp.float32).astype(jnp.int32)
    slot = (cs - 1) & 1

    eyef = (lax.broadcasted_iota(jnp.int32, (E, E), 0) ==
            lax.broadcasted_iota(jnp.int32, (E, E), 1)).astype(jnp.float32)
    nef = (cpad > 0).astype(jnp.float32)
    neT = lax.dot_general(nef, eyef, (((0,), (0,)), ((), ())))
    rowI = lax.broadcasted_iota(jnp.int32, (E, E), 0)
    colI = lax.broadcasted_iota(jnp.int32, (E, E), 1)
    cand = jnp.where((colI > rowI) & (neT > 0), colI, 99)
    nxt_e = jnp.min(cand, axis=1, keepdims=True)
    ohbe = (lax.broadcasted_iota(jnp.int32, (E, MMAX), 0) == be)
    nxtb = jnp.sum(jnp.where(ohbe, nxt_e, 0), axis=0, keepdims=True)

    be_n = jnp.where(colid == NACTO, nact, be)
    meta_ref[...] = jnp.concatenate([be_n, rs, slot, nxtb], axis=1)

    xlob = lax.bitcast_convert_type(xx[:, :D // 2], jnp.int32)
    xhib = lax.bitcast_convert_type(xx[:, D // 2:], jnp.int32)
    rb = jnp.int32(0x8000)
    xp_ref[...] = (lax.shift_right_logical(xlob + rb, 16) |
                   ((xhib + rb) & jnp.int32(-65536)))


def _router_call(flat, gate_w):
    return pl.pallas_call(
        _router_body,
        out_shape=(
            jax.ShapeDtypeStruct((TOPK, T), jnp.float32),
            jax.ShapeDtypeStruct((1, P), jnp.int32),
            jax.ShapeDtypeStruct((1, 4 * MMAX), jnp.int32),
            jax.ShapeDtypeStruct((T, D // 2), jnp.int32),
        ),
    )(flat, gate_w)


def _sc_mesh():
    return plsc.VectorSubcoreMesh(core_axis_name="c", subcore_axis_name="s")


def _dispatch_body(x_hbm, dest_hbm, xs_hbm, d0, d1, d2, d3, bufa, bufb, sem):
    wid = lax.axis_index("s") * NC + lax.axis_index("c")
    base = wid * PPW
    tbase = lax.rem(base, T)
    dbufs = [d0, d1, d2, d3]
    rbufs = [bufa, bufb]
    handles = []
    for i in range(NCHUNK):
        buf = rbufs[i % 2]
        if i >= 2:
            handles[i - 2].wait()
        pltpu.sync_copy(x_hbm.at[pl.ds(tbase + i * CH, CH)], buf)
        pltpu.sync_copy(dest_hbm.at[pl.ds(base + i * CH, CH)], dbufs[i])
        handles.append(pltpu.async_copy(buf, xs_hbm.at[dbufs[i]], sem))
    handles[-2].wait()
    handles[-1].wait()


def _dispatch_call(xp, dest):
    f = functools.partial(
        pl.kernel,
        mesh=_sc_mesh(),
        out_type=jax.ShapeDtypeStruct((NPAD, D // 2), jnp.int32),
        scratch_types=[
            pltpu.VMEM((CH,), jnp.int32), pltpu.VMEM((CH,), jnp.int32),
            pltpu.VMEM((CH,), jnp.int32), pltpu.VMEM((CH,), jnp.int32),
            pltpu.VMEM((CH, D // 2), jnp.int32), pltpu.VMEM((CH, D // 2), jnp.int32),
            pltpu.SemaphoreType.DMA,
        ],
    )(_dispatch_body)
    return f(xp, dest)


def _collect_body(eo_hbm, dest_hbm, geo_hbm,
                  d0, d1, d2, d3, bufa, bufb, bufc, gsem, wsem):
    wid = lax.axis_index("s") * NC + lax.axis_index("c")
    base = wid * PPW
    dbufs = [d0, d1, d2, d3]
    for i in range(NCHUNK):
        pltpu.sync_copy(dest_hbm.at[pl.ds(base + i * CH, CH)], dbufs[i])
    g0 = pltpu.async_copy(eo_hbm.at[d0], bufa, gsem)
    g1 = pltpu.async_copy(eo_hbm.at[d1], bufb, gsem)
    g2 = pltpu.async_copy(eo_hbm.at[d2], bufc, gsem)
    g0.wait()
    w0 = pltpu.async_copy(bufa, geo_hbm.at[pl.ds(base, CH)], wsem)
    g1.wait()
    w1 = pltpu.async_copy(bufb, geo_hbm.at[pl.ds(base + CH, CH)], wsem)
    g2.wait()
    w2 = pltpu.async_copy(bufc, geo_hbm.at[pl.ds(base + 2 * CH, CH)], wsem)
    w0.wait()
    g3 = pltpu.async_copy(eo_hbm.at[d3], bufa, gsem)
    g3.wait()
    w3 = pltpu.async_copy(bufa, geo_hbm.at[pl.ds(base + 3 * CH, CH)], wsem)
    w1.wait()
    w2.wait()
    w3.wait()


def _collect_call(eo, dest):
    f = functools.partial(
        pl.kernel,
        mesh=_sc_mesh(),
        out_type=jax.ShapeDtypeStruct((P, D // 2), jnp.int32),
        scratch_types=[
            pltpu.VMEM((CH,), jnp.int32), pltpu.VMEM((CH,), jnp.int32),
            pltpu.VMEM((CH,), jnp.int32), pltpu.VMEM((CH,), jnp.int32),
            pltpu.VMEM((CH, D // 2), jnp.int32), pltpu.VMEM((CH, D // 2), jnp.int32),
            pltpu.VMEM((CH, D // 2), jnp.int32),
            pltpu.SemaphoreType.DMA, pltpu.SemaphoreType.DMA,
        ],
    )(_collect_body)
    return f(eo, dest)


def _ffn_body(meta_ref, xs_ref, w1_any, w3_any, w2_any, out_ref,
              w1b0, w3b0, w2b0, w1b1, w3b1, w2b1, sem0, sem1):
    b = pl.program_id(0)
    nact = meta_ref[NACTO]

    def issue(e, bufs, sem):
        pltpu.make_async_copy(w1_any.at[e], bufs[0], sem).start()
        pltpu.make_async_copy(w3_any.at[e], bufs[1], sem).start()
        pltpu.make_async_copy(w2_any.at[e], bufs[2], sem).start()

    def drain(bufs, sem):
        pltpu.make_async_copy(w1_any.at[0], bufs[0], sem).wait()
        pltpu.make_async_copy(w3_any.at[0], bufs[1], sem).wait()
        pltpu.make_async_copy(w2_any.at[0], bufs[2], sem).wait()

    slot0 = (w1b0, w3b0, w2b0)
    slot1 = (w1b1, w3b1, w2b1)

    @pl.when(b == 0)
    def _():
        issue(meta_ref[0], slot0, sem0)
        nxt = meta_ref[NXTO]

        @pl.when(nxt < E)
        def _():
            issue(nxt, slot1, sem1)

        drain(slot0, sem0)

    @pl.when((b > 0) & (b < nact) & (meta_ref[RSO + b] == 1))
    def _():
        s = meta_ref[SLO + b]
        nxt = meta_ref[NXTO + b]

        @pl.when((nxt < E) & (s == 0))
        def _():
            issue(nxt, slot1, sem1)

        @pl.when((nxt < E) & (s == 1))
        def _():
            issue(nxt, slot0, sem0)

        @pl.when(s == 0)
        def _():
            drain(slot0, sem0)

        @pl.when(s == 1)
        def _():
            drain(slot1, sem1)

    def compute(bufs):
        xpk = xs_ref[...]
        xlo = lax.bitcast_convert_type(lax.shift_left(xpk, 16), jnp.float32)
        xhi = lax.bitcast_convert_type(xpk & jnp.int32(-65536), jnp.float32)
        xb = jnp.concatenate([xlo, xhi], axis=1)
        w1t = bufs[0][...]
        w3t = bufs[1][...]
        w2t = bufs[2][...]
        h1 = lax.dot_general(xb, w1t, (((1,), (1,)), ((), ())),
                             preferred_element_type=jnp.float32)
        h3 = lax.dot_general(xb, w3t, (((1,), (1,)), ((), ())),
                             preferred_element_type=jnp.float32)
        hh = h1 * (1.0 / (1.0 + jnp.exp(-h1))) * h3
        eo = lax.dot_general(hh, w2t, (((1,), (1,)), ((), ())),
                             preferred_element_type=jnp.float32)
        lob = lax.bitcast_convert_type(eo[:, :D // 2], jnp.int32)
        hib = lax.bitcast_convert_type(eo[:, D // 2:], jnp.int32)
        rb = jnp.int32(0x8000)
        lop = lax.shift_right_logical(lob + rb, 16)
        hip = (hib + rb) & jnp.int32(-65536)
        out_ref[...] = lop | hip

    @pl.when((b < nact) & (meta_ref[SLO + b] == 0))
    def _():
        compute(slot0)

    @pl.when((b < nact) & (meta_ref[SLO + b] == 1))
    def _():
        compute(slot1)


def _ffn_call(meta, xs, w1, w3, w2):
    grid_spec = pltpu.PrefetchScalarGridSpec(
        num_scalar_prefetch=1,
        grid=(NB,),
        in_specs=[
            pl.BlockSpec((BM, D // 2), lambda b, m: (b, 0)),
            pl.BlockSpec(memory_space=pl.ANY),
            pl.BlockSpec(memory_space=pl.ANY),
            pl.BlockSpec(memory_space=pl.ANY),
        ],
        out_specs=pl.BlockSpec((BM, D // 2), lambda b, m: (b, 0)),
        scratch_shapes=[
            pltpu.VMEM((H, D), jnp.float32), pltpu.VMEM((H, D), jnp.float32),
            pltpu.VMEM((D, H), jnp.float32),
            pltpu.VMEM((H, D), jnp.float32), pltpu.VMEM((H, D), jnp.float32),
            pltpu.VMEM((D, H), jnp.float32),
            pltpu.SemaphoreType.DMA, pltpu.SemaphoreType.DMA,
        ],
    )
    return pl.pallas_call(
        _ffn_body,
        grid_spec=grid_spec,
        out_shape=jax.ShapeDtypeStruct((NPAD, D // 2), jnp.int32),
        compiler_params=pltpu.CompilerParams(
            dimension_semantics=("arbitrary",),
        ),
    )(meta, xs, w1, w3, w2)


BT = 512


def _unpack_pair(g):
    lo = lax.bitcast_convert_type(lax.shift_left(g, 16), jnp.float32)
    hi = lax.bitcast_convert_type(g & jnp.int32(-65536), jnp.float32)
    return jnp.concatenate([lo, hi], axis=1)


def _combine_body(g0_ref, g1_ref, w_ref, out_ref):
    w = w_ref[...]
    w0 = jnp.transpose(w[0:1, :])
    w1 = jnp.transpose(w[1:2, :])
    g0 = _unpack_pair(g0_ref[...])
    g1 = _unpack_pair(g1_ref[...])
    out_ref[...] = g0 * w0 + g1 * w1


def _combine_call(geo, wts):
    return pl.pallas_call(
        _combine_body,
        grid=(T // BT,),
        in_specs=[
            pl.BlockSpec((BT, D // 2), lambda i: (i, 0)),
            pl.BlockSpec((BT, D // 2), lambda i: (i + T // BT, 0)),
            pl.BlockSpec((TOPK, BT), lambda i: (0, i)),
        ],
        out_specs=pl.BlockSpec((BT, D), lambda i: (i, 0)),
        out_shape=jax.ShapeDtypeStruct((T, D), jnp.float32),
    )(geo, geo, wts)


def kernel(x, gate_w, w1, w2, w3):
    B, T_, D_ = x.shape
    flat = x.reshape(T, D)
    wts, dest2, meta2, xp = _router_call(flat, gate_w)
    dest = dest2.reshape(P)
    meta = meta2.reshape(4 * MMAX)
    xs = _dispatch_call(xp, dest)
    eo = _ffn_call(meta, xs, w1, w3, w2)
    geo = _collect_call(eo, dest)
    out = _combine_call(geo, wts)
    return out.reshape(B, T_, D_)
